# baseline (speedup 1.0000x reference)
import functools

import jax
import jax.numpy as jnp
from jax import lax
from jax.experimental import pallas as pl
from jax.experimental.pallas import tpu as pltpu
from jax.experimental.pallas import tpu_sc as plsc

B = 16384
D = 32
NC = 2
NS = 16
NW = NC * NS
BPW = B // NW
CHUNK = 128
NCHUNK = BPW // CHUNK


def _sc_body(user_hbm, item_hbm, uf_hbm, if_hbm, out_hbm,
             uidx, iidx, urows, irows, sbuf, out_v, sem):
    wid = lax.axis_index("s") * NC + lax.axis_index("c")
    base = wid * BPW

    for k in range(NCHUNK):
        pltpu.sync_copy(user_hbm.at[pl.ds(base + k * CHUNK, CHUNK)],
                        uidx.at[k])
        pltpu.sync_copy(item_hbm.at[pl.ds(base + k * CHUNK, CHUNK)],
                        iidx.at[k])

    copies = []
    for k in range(NCHUNK):
        copies.append(pltpu.async_copy(
            uf_hbm.at[uidx.at[k]], urows.at[pl.ds(k * CHUNK, CHUNK)], sem))
        copies.append(pltpu.async_copy(
            if_hbm.at[iidx.at[k]], irows.at[pl.ds(k * CHUNK, CHUNK)], sem))
    for c in copies:
        c.wait()

    def fold(i, _):
        r0 = i * 4
        for rr in range(4):
            r = r0 + rr
            a = urows[r, pl.ds(0, 16)] * irows[r, pl.ds(0, 16)]
            b = urows[r, pl.ds(16, 16)] * irows[r, pl.ds(16, 16)]
            sbuf[pl.ds(r * 16, 16)] = a + b
        return 0

    lax.fori_loop(0, BPW // 4, fold, 0)

    lane = lax.broadcasted_iota(jnp.int32, (16,), 0)

    def group(g, _):
        base16 = g * 256 + lane * 16
        acc = jnp.zeros((16,), jnp.float32)
        for j in range(16):
            acc = acc + plsc.load_gather(sbuf, [base16 + j])
        out_v[pl.ds(g * 16, 16)] = acc
        return 0

    lax.fori_loop(0, BPW // 16, group, 0)

    pltpu.sync_copy(out_v, out_hbm.at[pl.ds(base, BPW)])


@jax.jit
def kernel(user, item, user_factors, item_factors):
    mesh = plsc.VectorSubcoreMesh(core_axis_name="c", subcore_axis_name="s")
    run = pl.kernel(
        _sc_body,
        out_type=jax.ShapeDtypeStruct((B,), jnp.float32),
        mesh=mesh,
        scratch_types=[
            pltpu.VMEM((NCHUNK, CHUNK), jnp.int32),
            pltpu.VMEM((NCHUNK, CHUNK), jnp.int32),
            pltpu.VMEM((BPW, D), jnp.float32),
            pltpu.VMEM((BPW, D), jnp.float32),
            pltpu.VMEM((BPW * 16,), jnp.float32),
            pltpu.VMEM((BPW,), jnp.float32),
            pltpu.SemaphoreType.DMA,
        ],
        compiler_params=pltpu.CompilerParams(
            needs_layout_passes=False, use_tc_tiling_on_sc=False),
    )
    return run(user.astype(jnp.int32), item.astype(jnp.int32),
               user_factors, item_factors)
